# Initial kernel scaffold; baseline (speedup 1.0000x reference)
import functools

import jax
import jax.numpy as jnp
from jax import lax
from jax.experimental import pallas as pl
from jax.experimental.pallas import tpu as pltpu
from jax.experimental.pallas import tpu_sc as plsc

NC = 2
NS = 16
LANES = 16


def _deg_body(n, dstr, ewr, out, acc, didx_v, ewv, zv, sem):
  c = lax.axis_index("c")
  s = lax.axis_index("s")
  wid = s * NC + c
  zero16 = jnp.zeros((LANES,), jnp.float32)

  @pl.when(s == 0)
  def _():
    @pl.loop(0, n // LANES)
    def _(i):
      zv[pl.ds(i * LANES, LANES)] = zero16
    pltpu.sync_copy(zv, acc)

  plsc.subcore_barrier()

  pltpu.sync_copy(dstr.at[wid], didx_v)
  pltpu.sync_copy(ewr.at[wid], ewv)
  nwin = didx_v.shape[0]
  wsz = didx_v.shape[1]

  @pl.loop(0, nwin)
  def _(w):
    off = pl.multiple_of(w * wsz, wsz)
    pltpu.async_copy(ewv.at[pl.ds(off, wsz)], acc.at[didx_v.at[w]], sem,
                     add=True).wait()

  plsc.subcore_barrier()

  @pl.when(s == 0)
  def _():
    pltpu.sync_copy(acc, out.at[c])


def _make_deg_kernel(n, e_pad):
  wsz = 64
  per_w = e_pad // (NC * NS)
  nwin = per_w // wsz
  mesh = plsc.VectorSubcoreMesh(core_axis_name="c", subcore_axis_name="s")
  return pl.kernel(
      functools.partial(_deg_body, n),
      out_type=jax.ShapeDtypeStruct((NC, n), jnp.float32),
      mesh=mesh,
      scratch_types=[
          pltpu.VMEM_SHARED((n,), jnp.float32),
          pltpu.VMEM((nwin, wsz), jnp.int32),
          pltpu.VMEM((per_w,), jnp.float32),
          pltpu.VMEM((n,), jnp.float32),
          pltpu.SemaphoreType.DMA,
      ],
  )


def _agg_body(n, f2, tbl, sidxr, dstr, ewr, out, acc, sidx_v, didx_v, ewv,
              msg, sem):
  c = lax.axis_index("c")
  s = lax.axis_index("s")
  zero16 = jnp.zeros((LANES,), jnp.float32)
  nwin = sidx_v.shape[0]
  wsz = sidx_v.shape[1]
  kf = f2 // LANES
  rows_per_tile = n // NS

  @pl.loop(0, wsz)
  def _(r):
    for k in range(kf):
      msg[r, pl.ds(k * LANES, LANES)] = zero16

  base = s * rows_per_tile
  nfull = rows_per_tile // wsz
  rem = rows_per_tile - nfull * wsz
  for z in range(nfull):
    pltpu.sync_copy(msg, acc.at[pl.ds(base + z * wsz, wsz)])
  if rem:
    pltpu.sync_copy(msg.at[pl.ds(0, rem)], acc.at[pl.ds(base + nfull * wsz,
                                                        rem)])
  plsc.subcore_barrier()

  pltpu.sync_copy(sidxr.at[s], sidx_v)
  pltpu.sync_copy(dstr.at[s], didx_v)
  pltpu.sync_copy(ewr.at[s], ewv)

  cn16 = jnp.full((LANES,), c * n, jnp.int32)

  @pl.loop(0, nwin)
  def _(r):
    for k in range(wsz // LANES):
      sl = pl.ds(k * LANES, LANES)
      sidx_v[r, sl] = sidx_v[r, sl] + cn16

  @pl.loop(0, nwin)
  def _(w):
    pltpu.async_copy(tbl.at[sidx_v.at[w]], msg, sem).wait()

    @pl.loop(0, wsz)
    def _(e):
      ew16 = plsc.load_gather(ewv, [jnp.full((LANES,), w * wsz + e,
                                             jnp.int32)])
      for k in range(kf):
        sl = pl.ds(k * LANES, LANES)
        msg[e, sl] = msg[e, sl] * ew16

    pltpu.async_copy(msg, acc.at[didx_v.at[w]], sem, add=True).wait()

  plsc.subcore_barrier()
  pltpu.sync_copy(acc.at[pl.ds(base, rows_per_tile)],
                  out.at[c, pl.ds(base, rows_per_tile)])


def _make_agg_kernel(n, e_pad, f2):
  wsz = 128
  per_s = e_pad // NS
  nwin = per_s // wsz
  mesh = plsc.VectorSubcoreMesh(core_axis_name="c", subcore_axis_name="s")
  return pl.kernel(
      functools.partial(_agg_body, n, f2),
      out_type=jax.ShapeDtypeStruct((NC, n, f2), jnp.float32),
      mesh=mesh,
      scratch_types=[
          pltpu.VMEM_SHARED((n, f2), jnp.float32),
          pltpu.VMEM((nwin, wsz), jnp.int32),
          pltpu.VMEM((nwin, wsz), jnp.int32),
          pltpu.VMEM((per_s,), jnp.float32),
          pltpu.VMEM((wsz, f2), jnp.float32),
          pltpu.SemaphoreType.DMA,
      ],
  )


ROWB = 512


def _mm1_body(x_ref, w1_ref, deg2_ref, hs_ref, dinv_ref):
  h = jnp.dot(x_ref[...], w1_ref[...], preferred_element_type=jnp.float32)
  deg = deg2_ref[0, :] + deg2_ref[1, :] + 1.0
  dinv = jnp.where(deg > 0, lax.rsqrt(deg), 0.0)
  hs = h * dinv[:, None]
  f2 = hs.shape[1] // 2
  hs_ref[...] = jnp.stack([hs[:, :f2], hs[:, f2:]])
  dinv_ref[...] = dinv


def _stage3_body(agg_ref, hs_ref, dinv_ref, b1_ref, w2_ref, hs2_ref):
  agg = jnp.concatenate([agg_ref[0], agg_ref[1]], axis=1)
  hs = jnp.concatenate([hs_ref[0], hs_ref[1]], axis=1)
  dinv = dinv_ref[...]
  z = jax.nn.relu(dinv[:, None] * (agg + hs) + b1_ref[0, :][None, :])
  h2 = jnp.dot(z, w2_ref[...], preferred_element_type=jnp.float32)
  hs2 = h2 * dinv[:, None]
  c2 = hs2.shape[1] // 2
  hs2_ref[...] = jnp.stack([hs2[:, :c2], hs2[:, c2:]])


def _stage5_body(agg_ref, hs2_ref, dinv_ref, b2_ref, out_ref):
  agg = jnp.concatenate([agg_ref[0], agg_ref[1]], axis=1)
  hs2 = jnp.concatenate([hs2_ref[0], hs2_ref[1]], axis=1)
  dinv = dinv_ref[...]
  logits = dinv[:, None] * (agg + hs2) + b2_ref[0, :][None, :]
  m = jnp.max(logits, axis=1, keepdims=True)
  lse = m + jnp.log(jnp.sum(jnp.exp(logits - m), axis=1, keepdims=True))
  out_ref[...] = logits - lse


def kernel(x, edge_index, edge_weight, W1, b1, W2, b2):
  n, f_in = x.shape
  hid = W1.shape[1]
  ncls = W2.shape[1]
  e = edge_index.shape[1]

  chunk = NS * 128
  e_pad = ((e + chunk - 1) // chunk) * chunk
  pad = e_pad - e
  src = edge_index[0]
  dst = edge_index[1]
  ew = edge_weight
  if pad:
    fill = (jnp.arange(pad, dtype=jnp.int32) * 37) % n
    src = jnp.concatenate([src, fill])
    dst = jnp.concatenate([dst, fill])
    ew = jnp.concatenate([ew, jnp.zeros((pad,), ew.dtype)])

  per_w = e_pad // (NC * NS)
  dst_deg = dst.reshape(NC * NS, per_w // 64, 64)
  ew_deg = ew.reshape(NC * NS, per_w)

  per_s = e_pad // NS
  src_agg = src.reshape(NS, per_s // 128, 128)
  dst_agg = dst.reshape(NS, per_s // 128, 128)
  ew_agg = ew.reshape(NS, per_s)

  deg2 = _make_deg_kernel(n, e_pad)(dst_deg, ew_deg)

  grid = (n + ROWB - 1) // ROWB
  hs_r, dinv = pl.pallas_call(
      _mm1_body,
      grid=(grid,),
      in_specs=[
          pl.BlockSpec((ROWB, f_in), lambda i: (i, 0)),
          pl.BlockSpec((f_in, hid), lambda i: (0, 0)),
          pl.BlockSpec((NC, ROWB), lambda i: (0, i)),
      ],
      out_specs=[
          pl.BlockSpec((NC, ROWB, hid // 2), lambda i: (0, i, 0)),
          pl.BlockSpec((ROWB,), lambda i: (i,)),
      ],
      out_shape=[
          jax.ShapeDtypeStruct((NC, n, hid // 2), jnp.float32),
          jax.ShapeDtypeStruct((n,), jnp.float32),
      ],
  )(x, W1, deg2)

  tbl1 = hs_r.reshape(NC * n, hid // 2)
  agg1 = _make_agg_kernel(n, e_pad, hid // 2)(tbl1, src_agg, dst_agg, ew_agg)

  hs2_r = pl.pallas_call(
      _stage3_body,
      grid=(grid,),
      in_specs=[
          pl.BlockSpec((NC, ROWB, hid // 2), lambda i: (0, i, 0)),
          pl.BlockSpec((NC, ROWB, hid // 2), lambda i: (0, i, 0)),
          pl.BlockSpec((ROWB,), lambda i: (i,)),
          pl.BlockSpec((1, hid), lambda i: (0, 0)),
          pl.BlockSpec((hid, ncls), lambda i: (0, 0)),
      ],
      out_specs=pl.BlockSpec((NC, ROWB, ncls // 2), lambda i: (0, i, 0)),
      out_shape=jax.ShapeDtypeStruct((NC, n, ncls // 2), jnp.float32),
  )(agg1, hs_r, dinv, b1.reshape(1, hid), W2)

  tbl2 = hs2_r.reshape(NC * n, ncls // 2)
  agg2 = _make_agg_kernel(n, e_pad, ncls // 2)(tbl2, src_agg, dst_agg, ew_agg)

  out = pl.pallas_call(
      _stage5_body,
      grid=(grid,),
      in_specs=[
          pl.BlockSpec((NC, ROWB, ncls // 2), lambda i: (0, i, 0)),
          pl.BlockSpec((NC, ROWB, ncls // 2), lambda i: (0, i, 0)),
          pl.BlockSpec((ROWB,), lambda i: (i,)),
          pl.BlockSpec((1, ncls), lambda i: (0, 0)),
      ],
      out_specs=pl.BlockSpec((ROWB, ncls), lambda i: (i, 0)),
      out_shape=jax.ShapeDtypeStruct((n, ncls), jnp.float32),
  )(agg2, hs2_r, dinv, b2.reshape(1, ncls))

  return out


# speedup vs baseline: 10.2390x; 10.2390x over previous
import functools

import jax
import jax.numpy as jnp
from jax import lax
from jax.experimental import pallas as pl
from jax.experimental.pallas import tpu as pltpu
from jax.experimental.pallas import tpu_sc as plsc

NC = 2
NS = 16
LANES = 16


def _deg_body(n, dstr, ewr, out, acc, didx_v, ewv, zv, sem):
  c = lax.axis_index("c")
  s = lax.axis_index("s")
  wid = s * NC + c
  zero16 = jnp.zeros((LANES,), jnp.float32)

  @pl.when(s == 0)
  def _():
    @pl.loop(0, n // LANES)
    def _(i):
      zv[pl.ds(i * LANES, LANES)] = zero16
    pltpu.sync_copy(zv, acc)

  plsc.subcore_barrier()

  pltpu.sync_copy(dstr.at[wid], didx_v)
  pltpu.sync_copy(ewr.at[wid], ewv)
  nwin = didx_v.shape[0]
  wsz = didx_v.shape[1]

  @pl.loop(0, nwin)
  def _(w):
    off = pl.multiple_of(w * wsz, wsz)
    pltpu.async_copy(ewv.at[pl.ds(off, wsz)], acc.at[didx_v.at[w]], sem,
                     add=True).wait()

  plsc.subcore_barrier()

  @pl.when(s == 0)
  def _():
    pltpu.sync_copy(acc, out.at[c])


def _make_deg_kernel(n, e_pad):
  wsz = 64
  per_w = e_pad // (NC * NS)
  nwin = per_w // wsz
  mesh = plsc.VectorSubcoreMesh(core_axis_name="c", subcore_axis_name="s")
  return pl.kernel(
      functools.partial(_deg_body, n),
      out_type=jax.ShapeDtypeStruct((NC, n), jnp.float32),
      mesh=mesh,
      compiler_params=pltpu.CompilerParams(needs_layout_passes=False),
      scratch_types=[
          pltpu.VMEM_SHARED((n,), jnp.float32),
          pltpu.VMEM((nwin, wsz), jnp.int32),
          pltpu.VMEM((per_w,), jnp.float32),
          pltpu.VMEM((n,), jnp.float32),
          pltpu.SemaphoreType.DMA,
      ],
  )


def _agg_body(n, core_split, tbl, sidxr, dstr, ewr, out, acc, sidx_v, didx_v,
              ewv, msg, sem):
  c = lax.axis_index("c")
  s = lax.axis_index("s")
  zero16 = jnp.zeros((LANES,), jnp.float32)
  nwin = sidx_v.shape[0]
  wsz = sidx_v.shape[1]
  kf = 128 // LANES
  gid = s if core_split else s * NC + c
  rpt = ((n + NS - 1) // NS + 7) // 8 * 8

  @pl.loop(0, wsz)
  def _(r):
    for k in range(kf):
      msg[r, pl.ds(k * LANES, LANES)] = zero16

  for t in range(NS):
    base = t * rpt
    cnt = min(rpt, n - base)
    if cnt <= 0:
      continue

    @pl.when(s == t)
    def _(base=base, cnt=cnt):
      nfull = cnt // wsz
      rem = cnt - nfull * wsz
      for z in range(nfull):
        pltpu.sync_copy(msg, acc.at[pl.ds(base + z * wsz, wsz)])
      if rem:
        pltpu.sync_copy(msg.at[pl.ds(0, rem)],
                        acc.at[pl.ds(base + nfull * wsz, rem)])

  plsc.subcore_barrier()

  pltpu.sync_copy(sidxr.at[gid], sidx_v)
  pltpu.sync_copy(dstr.at[gid], didx_v)
  pltpu.sync_copy(ewr.at[gid], ewv)

  if core_split:
    cn16 = jnp.full((LANES,), c * n, jnp.int32)

    @pl.loop(0, nwin)
    def _(r):
      for k in range(wsz // LANES):
        sl = pl.ds(k * LANES, LANES)
        sidx_v[r, sl] = sidx_v[r, sl] + cn16

  @pl.loop(0, nwin)
  def _(w):
    pltpu.async_copy(tbl.at[sidx_v.at[w]], msg, sem).wait()

    @pl.loop(0, wsz)
    def _(e):
      ew16 = plsc.load_gather(ewv, [jnp.full((LANES,), w * wsz + e,
                                             jnp.int32)])
      for k in range(kf):
        sl = pl.ds(k * LANES, LANES)
        msg[e, sl] = msg[e, sl] * ew16

    pltpu.async_copy(msg, acc.at[didx_v.at[w]], sem, add=True).wait()

  plsc.subcore_barrier()
  for t in range(NS):
    base = t * rpt
    cnt = min(rpt, n - base)
    if cnt <= 0:
      continue

    @pl.when(s == t)
    def _(base=base, cnt=cnt):
      pltpu.sync_copy(acc.at[pl.ds(base, cnt)], out.at[c, pl.ds(base, cnt)])


def _make_agg_kernel(n, e_pad, core_split):
  wsz = 128
  ngroups = NS if core_split else NS * NC
  per_g = e_pad // ngroups
  nwin = per_g // wsz
  mesh = plsc.VectorSubcoreMesh(core_axis_name="c", subcore_axis_name="s")
  return pl.kernel(
      functools.partial(_agg_body, n, core_split),
      out_type=jax.ShapeDtypeStruct((NC, n, 128), jnp.float32),
      mesh=mesh,
      compiler_params=pltpu.CompilerParams(needs_layout_passes=False),
      scratch_types=[
          pltpu.VMEM_SHARED((n, 128), jnp.float32),
          pltpu.VMEM((nwin, wsz), jnp.int32),
          pltpu.VMEM((nwin, wsz), jnp.int32),
          pltpu.VMEM((per_g,), jnp.float32),
          pltpu.VMEM((wsz, 128), jnp.float32),
          pltpu.SemaphoreType.DMA,
      ],
  )


ROWB = 512


def _mm1_body(x_ref, w1_ref, deg2_ref, hs_ref, dinv_ref):
  h = jnp.dot(x_ref[...], w1_ref[...], preferred_element_type=jnp.float32)
  deg = deg2_ref[0, :] + deg2_ref[1, :] + 1.0
  dinv = jnp.where(deg > 0, lax.rsqrt(deg), 0.0)
  hs = h * dinv[:, None]
  f2 = hs.shape[1] // 2
  hs_ref[...] = jnp.stack([hs[:, :f2], hs[:, f2:]])
  dinv_ref[...] = dinv


def _stage3_body(agg_ref, hs_ref, dinv_ref, b1_ref, w2_ref, hs2_ref):
  agg = jnp.concatenate([agg_ref[0], agg_ref[1]], axis=1)
  hs = jnp.concatenate([hs_ref[0], hs_ref[1]], axis=1)
  dinv = dinv_ref[...]
  z = jax.nn.relu(dinv[:, None] * (agg + hs) + b1_ref[0, :][None, :])
  h2 = jnp.dot(z, w2_ref[...], preferred_element_type=jnp.float32)
  hs2 = h2 * dinv[:, None]
  hs2_ref[...] = jnp.concatenate(
      [hs2, jnp.zeros_like(hs2)], axis=1)


def _stage5_body(agg_ref, hs2_ref, dinv_ref, b2_ref, out_ref):
  ncls = out_ref.shape[1]
  agg = agg_ref[0, :, :ncls] + agg_ref[1, :, :ncls]
  hs2 = hs2_ref[:, :ncls]
  dinv = dinv_ref[...]
  logits = dinv[:, None] * (agg + hs2) + b2_ref[0, :][None, :]
  m = jnp.max(logits, axis=1, keepdims=True)
  lse = m + jnp.log(jnp.sum(jnp.exp(logits - m), axis=1, keepdims=True))
  out_ref[...] = logits - lse


def kernel(x, edge_index, edge_weight, W1, b1, W2, b2):
  n, f_in = x.shape
  hid = W1.shape[1]
  ncls = W2.shape[1]
  e = edge_index.shape[1]

  chunk = NC * NS * 128
  e_pad = ((e + chunk - 1) // chunk) * chunk
  pad = e_pad - e
  src = edge_index[0]
  dst = edge_index[1]
  ew = edge_weight
  if pad:
    fill = (jnp.arange(pad, dtype=jnp.int32) * 37) % n
    src = jnp.concatenate([src, fill])
    dst = jnp.concatenate([dst, fill])
    ew = jnp.concatenate([ew, jnp.zeros((pad,), ew.dtype)])

  per_w = e_pad // (NC * NS)
  dst_deg = dst.reshape(NC * NS, per_w // 64, 64)
  ew_deg = ew.reshape(NC * NS, per_w)

  per_s = e_pad // NS
  src_agg1 = src.reshape(NS, per_s // 128, 128)
  dst_agg1 = dst.reshape(NS, per_s // 128, 128)
  ew_agg1 = ew.reshape(NS, per_s)

  src_agg2 = src.reshape(NC * NS, per_w // 128, 128)
  dst_agg2 = dst.reshape(NC * NS, per_w // 128, 128)
  ew_agg2 = ew.reshape(NC * NS, per_w)

  deg2 = _make_deg_kernel(n, e_pad)(dst_deg, ew_deg)

  grid = (n + ROWB - 1) // ROWB
  hs_r, dinv = pl.pallas_call(
      _mm1_body,
      grid=(grid,),
      in_specs=[
          pl.BlockSpec((ROWB, f_in), lambda i: (i, 0)),
          pl.BlockSpec((f_in, hid), lambda i: (0, 0)),
          pl.BlockSpec((NC, ROWB), lambda i: (0, i)),
      ],
      out_specs=[
          pl.BlockSpec((NC, ROWB, hid // 2), lambda i: (0, i, 0)),
          pl.BlockSpec((ROWB,), lambda i: (i,)),
      ],
      out_shape=[
          jax.ShapeDtypeStruct((NC, n, hid // 2), jnp.float32),
          jax.ShapeDtypeStruct((n,), jnp.float32),
      ],
  )(x, W1, deg2)

  tbl1 = hs_r.reshape(NC * n, hid // 2)
  agg1 = _make_agg_kernel(n, e_pad, True)(tbl1, src_agg1, dst_agg1, ew_agg1)

  hs2p = pl.pallas_call(
      _stage3_body,
      grid=(grid,),
      in_specs=[
          pl.BlockSpec((NC, ROWB, hid // 2), lambda i: (0, i, 0)),
          pl.BlockSpec((NC, ROWB, hid // 2), lambda i: (0, i, 0)),
          pl.BlockSpec((ROWB,), lambda i: (i,)),
          pl.BlockSpec((1, hid), lambda i: (0, 0)),
          pl.BlockSpec((hid, ncls), lambda i: (0, 0)),
      ],
      out_specs=pl.BlockSpec((ROWB, 2 * ncls), lambda i: (i, 0)),
      out_shape=jax.ShapeDtypeStruct((n, 2 * ncls), jnp.float32),
  )(agg1, hs_r, dinv, b1.reshape(1, hid), W2)

  agg2 = _make_agg_kernel(n, e_pad, False)(hs2p, src_agg2, dst_agg2, ew_agg2)

  out = pl.pallas_call(
      _stage5_body,
      grid=(grid,),
      in_specs=[
          pl.BlockSpec((NC, ROWB, 2 * ncls), lambda i: (0, i, 0)),
          pl.BlockSpec((ROWB, 2 * ncls), lambda i: (i, 0)),
          pl.BlockSpec((ROWB,), lambda i: (i,)),
          pl.BlockSpec((1, ncls), lambda i: (0, 0)),
      ],
      out_specs=pl.BlockSpec((ROWB, ncls), lambda i: (i, 0)),
      out_shape=jax.ShapeDtypeStruct((n, ncls), jnp.float32),
  )(agg2, hs2p, dinv, b2.reshape(1, ncls))

  return out


# speedup vs baseline: 14.2089x; 1.3877x over previous
import functools

import jax
import jax.numpy as jnp
from jax import lax
from jax.experimental import pallas as pl
from jax.experimental.pallas import tpu as pltpu
from jax.experimental.pallas import tpu_sc as plsc

NC = 2
NS = 16
LANES = 16


def _deg_body(n, dstr, ewr, out, acc, didx_v, ewv, zv, sem):
  c = lax.axis_index("c")
  s = lax.axis_index("s")
  wid = s * NC + c
  zero16 = jnp.zeros((LANES,), jnp.float32)

  @pl.when(s == 0)
  def _():
    @pl.loop(0, n // LANES)
    def _(i):
      zv[pl.ds(i * LANES, LANES)] = zero16
    pltpu.sync_copy(zv, acc)

  plsc.subcore_barrier()

  pltpu.sync_copy(dstr.at[wid], didx_v)
  pltpu.sync_copy(ewr.at[wid], ewv)
  nwin = didx_v.shape[0]
  wsz = didx_v.shape[1]

  @pl.loop(0, nwin)
  def _(w):
    off = pl.multiple_of(w * wsz, wsz)
    pltpu.async_copy(ewv.at[pl.ds(off, wsz)], acc.at[didx_v.at[w]], sem,
                     add=True).wait()

  plsc.subcore_barrier()

  @pl.when(s == 0)
  def _():
    pltpu.sync_copy(acc, out.at[c])


def _make_deg_kernel(n, e_pad):
  wsz = 64
  per_w = e_pad // (NC * NS)
  nwin = per_w // wsz
  mesh = plsc.VectorSubcoreMesh(core_axis_name="c", subcore_axis_name="s")
  return pl.kernel(
      functools.partial(_deg_body, n),
      out_type=jax.ShapeDtypeStruct((NC, n), jnp.float32),
      mesh=mesh,
      compiler_params=pltpu.CompilerParams(needs_layout_passes=False),
      scratch_types=[
          pltpu.VMEM_SHARED((n,), jnp.float32),
          pltpu.VMEM((nwin, wsz), jnp.int32),
          pltpu.VMEM((per_w,), jnp.float32),
          pltpu.VMEM((n,), jnp.float32),
          pltpu.SemaphoreType.DMA,
      ],
  )


GRP = 8


def _agg_body(n, core_split, kf_scale, nwin_total, tbl, sidxr, dstr, ewr, out,
              acc, sidx_v, didx_v, ewv, msga, msgb, semga, semgb, semsa,
              semsb):
  c = lax.axis_index("c")
  s = lax.axis_index("s")
  zero16 = jnp.zeros((LANES,), jnp.float32)
  nwin = nwin_total
  wsz = sidx_v.shape[1]
  kf = 128 // LANES
  gid = s if core_split else s * NC + c
  n_pad = acc.shape[0]
  rpt = n_pad // NS

  @pl.loop(0, wsz)
  def _(r):
    for k in range(kf):
      msga[r, pl.ds(k * LANES, LANES)] = zero16

  nfull = rpt // wsz
  rem = rpt - nfull * wsz
  base = s * rpt
  for z in range(nfull):
    pltpu.sync_copy(msga, acc.at[pl.ds(base + z * wsz, wsz)])
  if rem:
    pltpu.sync_copy(msga.at[pl.ds(0, rem)],
                    acc.at[pl.ds(base + nfull * wsz, rem)])

  plsc.subcore_barrier()

  def fire_gather(w, buf, sem):
    pltpu.async_copy(tbl.at[sidx_v.at[w]], buf, sem)

  def wait_gather(w, buf, sem):
    pltpu.make_async_copy(tbl.at[sidx_v.at[w]], buf, sem).wait()

  def fire_scatter(w, buf, sem):
    pltpu.async_copy(buf, acc.at[didx_v.at[w]], sem, add=True)

  def wait_scatter(w, buf, sem):
    pltpu.make_async_copy(buf, acc.at[didx_v.at[w]], sem).wait()

  def scale(j, buf):
    @pl.loop(0, wsz, unroll=4)
    def _(e):
      ew16 = plsc.load_gather(ewv, [jnp.full((LANES,), j * wsz + e,
                                             jnp.int32)])
      for k in range(kf_scale):
        sl = pl.ds(k * LANES, LANES)
        buf[e, sl] = buf[e, sl] * ew16

  bufs = (msga, msgb)
  gsems = (semga, semgb)
  ssems = (semsa, semsb)
  cn16 = jnp.full((LANES,), c * n, jnp.int32)

  @pl.loop(0, nwin // GRP)
  def _(g):
    w0 = pl.multiple_of(g * GRP, GRP)
    pltpu.sync_copy(sidxr.at[gid, pl.ds(w0, GRP)], sidx_v)
    pltpu.sync_copy(dstr.at[gid, pl.ds(w0, GRP)], didx_v)
    pltpu.sync_copy(ewr.at[gid, pl.ds(w0 * wsz, GRP * wsz)], ewv)
    if core_split:
      @pl.loop(0, GRP)
      def _(r):
        for k in range(wsz // LANES):
          sl = pl.ds(k * LANES, LANES)
          sidx_v[r, sl] = sidx_v[r, sl] + cn16

    fire_gather(0, msga, semga)
    for j in range(GRP):
      cur, nxt = bufs[j % 2], bufs[1 - j % 2]
      gcur, gnxt = gsems[j % 2], gsems[1 - j % 2]
      scur, snxt = ssems[j % 2], ssems[1 - j % 2]
      wait_gather(j, cur, gcur)
      if j >= 1:
        wait_scatter(j - 1, nxt, snxt)
      if j < GRP - 1:
        fire_gather(j + 1, nxt, gnxt)
      scale(j, cur)
      fire_scatter(j, cur, scur)
    wait_scatter(GRP - 1, bufs[(GRP - 1) % 2], ssems[(GRP - 1) % 2])

  plsc.subcore_barrier()
  pltpu.sync_copy(acc.at[pl.ds(base, rpt)], out.at[c, pl.ds(base, rpt)])


def _make_agg_kernel(n, e_pad, core_split, kf_scale):
  wsz = 128
  ngroups = NS if core_split else NS * NC
  per_g = e_pad // ngroups
  nwin = per_g // wsz
  n_pad = ((n + NS * 8 - 1) // (NS * 8)) * NS * 8
  mesh = plsc.VectorSubcoreMesh(core_axis_name="c", subcore_axis_name="s")
  return pl.kernel(
      functools.partial(_agg_body, n, core_split, kf_scale, nwin),
      out_type=jax.ShapeDtypeStruct((NC, n_pad, 128), jnp.float32),
      mesh=mesh,
      compiler_params=pltpu.CompilerParams(needs_layout_passes=False),
      scratch_types=[
          pltpu.VMEM_SHARED((n_pad, 128), jnp.float32),
          pltpu.VMEM((GRP, wsz), jnp.int32),
          pltpu.VMEM((GRP, wsz), jnp.int32),
          pltpu.VMEM((GRP * wsz,), jnp.float32),
          pltpu.VMEM((wsz, 128), jnp.float32),
          pltpu.VMEM((wsz, 128), jnp.float32),
          pltpu.SemaphoreType.DMA,
          pltpu.SemaphoreType.DMA,
          pltpu.SemaphoreType.DMA,
          pltpu.SemaphoreType.DMA,
      ],
  )


ROWB = 512


def _mm1_body(x_ref, w1_ref, deg2_ref, hs_ref, dinv_ref):
  h = jnp.dot(x_ref[...], w1_ref[...], preferred_element_type=jnp.float32)
  deg = deg2_ref[0, :] + deg2_ref[1, :] + 1.0
  dinv = jnp.where(deg > 0, lax.rsqrt(deg), 0.0)
  hs = h * dinv[:, None]
  f2 = hs.shape[1] // 2
  hs_ref[...] = jnp.stack([hs[:, :f2], hs[:, f2:]])
  dinv_ref[...] = dinv


def _stage3_body(agg_ref, hs_ref, dinv_ref, b1_ref, w2_ref, hs2_ref):
  agg = jnp.concatenate([agg_ref[0], agg_ref[1]], axis=1)
  hs = jnp.concatenate([hs_ref[0], hs_ref[1]], axis=1)
  dinv = dinv_ref[...]
  z = jax.nn.relu(dinv[:, None] * (agg + hs) + b1_ref[0, :][None, :])
  h2 = jnp.dot(z, w2_ref[...], preferred_element_type=jnp.float32)
  hs2 = h2 * dinv[:, None]
  hs2_ref[...] = jnp.concatenate(
      [hs2, jnp.zeros_like(hs2)], axis=1)


def _stage5_body(agg_ref, hs2_ref, dinv_ref, b2_ref, out_ref):
  ncls = out_ref.shape[1]
  agg = agg_ref[0, :, :ncls] + agg_ref[1, :, :ncls]
  hs2 = hs2_ref[:, :ncls]
  dinv = dinv_ref[...]
  logits = dinv[:, None] * (agg + hs2) + b2_ref[0, :][None, :]
  m = jnp.max(logits, axis=1, keepdims=True)
  lse = m + jnp.log(jnp.sum(jnp.exp(logits - m), axis=1, keepdims=True))
  out_ref[...] = logits - lse


def kernel(x, edge_index, edge_weight, W1, b1, W2, b2):
  n, f_in = x.shape
  hid = W1.shape[1]
  ncls = W2.shape[1]
  e = edge_index.shape[1]

  chunk = NC * NS * 128
  e_pad = ((e + chunk - 1) // chunk) * chunk
  pad = e_pad - e
  src = edge_index[0]
  dst = edge_index[1]
  ew = edge_weight
  if pad:
    fill = (jnp.arange(pad, dtype=jnp.int32) * 37) % n
    src = jnp.concatenate([src, fill])
    dst = jnp.concatenate([dst, fill])
    ew = jnp.concatenate([ew, jnp.zeros((pad,), ew.dtype)])

  per_w = e_pad // (NC * NS)
  dst_deg = dst.reshape(NC * NS, per_w // 64, 64)
  ew_deg = ew.reshape(NC * NS, per_w)

  per_s = e_pad // NS
  src_agg1 = src.reshape(NS, per_s // 128, 128)
  dst_agg1 = dst.reshape(NS, per_s // 128, 128)
  ew_agg1 = ew.reshape(NS, per_s)

  src_agg2 = src.reshape(NC * NS, per_w // 128, 128)
  dst_agg2 = dst.reshape(NC * NS, per_w // 128, 128)
  ew_agg2 = ew.reshape(NC * NS, per_w)

  deg2 = _make_deg_kernel(n, e_pad)(dst_deg, ew_deg)

  grid = (n + ROWB - 1) // ROWB
  hs_r, dinv = pl.pallas_call(
      _mm1_body,
      grid=(grid,),
      in_specs=[
          pl.BlockSpec((ROWB, f_in), lambda i: (i, 0)),
          pl.BlockSpec((f_in, hid), lambda i: (0, 0)),
          pl.BlockSpec((NC, ROWB), lambda i: (0, i)),
      ],
      out_specs=[
          pl.BlockSpec((NC, ROWB, hid // 2), lambda i: (0, i, 0)),
          pl.BlockSpec((ROWB,), lambda i: (i,)),
      ],
      out_shape=[
          jax.ShapeDtypeStruct((NC, n, hid // 2), jnp.float32),
          jax.ShapeDtypeStruct((n,), jnp.float32),
      ],
  )(x, W1, deg2)

  tbl1 = hs_r.reshape(NC * n, hid // 2)
  agg1 = _make_agg_kernel(n, e_pad, True, 8)(tbl1, src_agg1, dst_agg1,
                                             ew_agg1)

  hs2p = pl.pallas_call(
      _stage3_body,
      grid=(grid,),
      in_specs=[
          pl.BlockSpec((NC, ROWB, hid // 2), lambda i: (0, i, 0)),
          pl.BlockSpec((NC, ROWB, hid // 2), lambda i: (0, i, 0)),
          pl.BlockSpec((ROWB,), lambda i: (i,)),
          pl.BlockSpec((1, hid), lambda i: (0, 0)),
          pl.BlockSpec((hid, ncls), lambda i: (0, 0)),
      ],
      out_specs=pl.BlockSpec((ROWB, 2 * ncls), lambda i: (i, 0)),
      out_shape=jax.ShapeDtypeStruct((n, 2 * ncls), jnp.float32),
  )(agg1, hs_r, dinv, b1.reshape(1, hid), W2)

  agg2 = _make_agg_kernel(n, e_pad, False, 4)(hs2p, src_agg2, dst_agg2,
                                              ew_agg2)

  out = pl.pallas_call(
      _stage5_body,
      grid=(grid,),
      in_specs=[
          pl.BlockSpec((NC, ROWB, 2 * ncls), lambda i: (0, i, 0)),
          pl.BlockSpec((ROWB, 2 * ncls), lambda i: (i, 0)),
          pl.BlockSpec((ROWB,), lambda i: (i,)),
          pl.BlockSpec((1, ncls), lambda i: (0, 0)),
      ],
      out_specs=pl.BlockSpec((ROWB, ncls), lambda i: (i, 0)),
      out_shape=jax.ShapeDtypeStruct((n, ncls), jnp.float32),
  )(agg2, hs2p, dinv, b2.reshape(1, ncls))

  return out


# speedup vs baseline: 14.6063x; 1.0280x over previous
import functools

import jax
import jax.numpy as jnp
from jax import lax
from jax.experimental import pallas as pl
from jax.experimental.pallas import tpu as pltpu
from jax.experimental.pallas import tpu_sc as plsc

NC = 2
NS = 16
LANES = 16


def _deg_body(n, dstr, ewr, out, acc, didx_v, ewv, zv, sem):
  c = lax.axis_index("c")
  s = lax.axis_index("s")
  wid = s * NC + c
  zero16 = jnp.zeros((LANES,), jnp.float32)

  @pl.when(s == 0)
  def _():
    @pl.loop(0, n // LANES)
    def _(i):
      zv[pl.ds(i * LANES, LANES)] = zero16
    pltpu.sync_copy(zv, acc)

  plsc.subcore_barrier()

  pltpu.sync_copy(dstr.at[wid], didx_v)
  pltpu.sync_copy(ewr.at[wid], ewv)
  nwin = didx_v.shape[0]
  wsz = didx_v.shape[1]

  @pl.loop(0, nwin // 8)
  def _(g):
    w0 = pl.multiple_of(g * 8, 8)
    for i in range(8):
      off = pl.multiple_of((w0 + i) * wsz, wsz)
      pltpu.async_copy(ewv.at[pl.ds(off, wsz)], acc.at[didx_v.at[w0 + i]],
                       sem, add=True)
    for i in range(8):
      off = pl.multiple_of((w0 + i) * wsz, wsz)
      pltpu.make_async_copy(ewv.at[pl.ds(off, wsz)],
                            acc.at[didx_v.at[w0 + i]], sem).wait()

  plsc.subcore_barrier()

  @pl.when(s == 0)
  def _():
    pltpu.sync_copy(acc, out.at[c])


def _make_deg_kernel(n, e_pad):
  wsz = 64
  per_w = e_pad // (NC * NS)
  nwin = per_w // wsz
  mesh = plsc.VectorSubcoreMesh(core_axis_name="c", subcore_axis_name="s")
  return pl.kernel(
      functools.partial(_deg_body, n),
      out_type=jax.ShapeDtypeStruct((NC, n), jnp.float32),
      mesh=mesh,
      compiler_params=pltpu.CompilerParams(needs_layout_passes=False),
      scratch_types=[
          pltpu.VMEM_SHARED((n,), jnp.float32),
          pltpu.VMEM((nwin, wsz), jnp.int32),
          pltpu.VMEM((per_w,), jnp.float32),
          pltpu.VMEM((n,), jnp.float32),
          pltpu.SemaphoreType.DMA,
      ],
  )


GRP = 8


def _agg_body(n, core_split, kf_scale, nwin_total, tbl, sidxr, dstr, ewr, out,
              acc, sidx_v, didx_v, ewv, msga, msgb, semga, semgb, semsa,
              semsb):
  c = lax.axis_index("c")
  s = lax.axis_index("s")
  zero16 = jnp.zeros((LANES,), jnp.float32)
  nwin = nwin_total
  wsz = sidx_v.shape[1]
  kf = 128 // LANES
  gid = s if core_split else s * NC + c
  n_pad = acc.shape[0]
  rpt = n_pad // NS

  @pl.loop(0, wsz)
  def _(r):
    for k in range(kf):
      msga[r, pl.ds(k * LANES, LANES)] = zero16

  nfull = rpt // wsz
  rem = rpt - nfull * wsz
  base = s * rpt
  for z in range(nfull):
    pltpu.sync_copy(msga, acc.at[pl.ds(base + z * wsz, wsz)])
  if rem:
    pltpu.sync_copy(msga.at[pl.ds(0, rem)],
                    acc.at[pl.ds(base + nfull * wsz, rem)])

  plsc.subcore_barrier()

  def fire_gather(w, buf, sem):
    pltpu.async_copy(tbl.at[sidx_v.at[w]], buf, sem)

  def wait_gather(w, buf, sem):
    pltpu.make_async_copy(tbl.at[sidx_v.at[w]], buf, sem).wait()

  def fire_scatter(w, buf, sem):
    pltpu.async_copy(buf, acc.at[didx_v.at[w]], sem, add=True)

  def wait_scatter(w, buf, sem):
    pltpu.make_async_copy(buf, acc.at[didx_v.at[w]], sem).wait()

  def scale(j, buf):
    @pl.loop(0, wsz, unroll=8)
    def _(e):
      ew16 = plsc.load_gather(ewv, [jnp.full((LANES,), j * wsz + e,
                                             jnp.int32)])
      for k in range(kf_scale):
        sl = pl.ds(k * LANES, LANES)
        buf[e, sl] = buf[e, sl] * ew16

  bufs = (msga, msgb)
  gsems = (semga, semgb)
  ssems = (semsa, semsb)
  cn16 = jnp.full((LANES,), c * n, jnp.int32)

  @pl.loop(0, nwin // GRP)
  def _(g):
    w0 = pl.multiple_of(g * GRP, GRP)
    pltpu.sync_copy(sidxr.at[gid, pl.ds(w0, GRP)], sidx_v)
    pltpu.sync_copy(dstr.at[gid, pl.ds(w0, GRP)], didx_v)
    pltpu.sync_copy(ewr.at[gid, pl.ds(w0 * wsz, GRP * wsz)], ewv)
    if core_split:
      @pl.loop(0, GRP)
      def _(r):
        for k in range(wsz // LANES):
          sl = pl.ds(k * LANES, LANES)
          sidx_v[r, sl] = sidx_v[r, sl] + cn16

    fire_gather(0, msga, semga)
    for j in range(GRP):
      cur, nxt = bufs[j % 2], bufs[1 - j % 2]
      gcur, gnxt = gsems[j % 2], gsems[1 - j % 2]
      scur, snxt = ssems[j % 2], ssems[1 - j % 2]
      wait_gather(j, cur, gcur)
      if j >= 1:
        wait_scatter(j - 1, nxt, snxt)
      if j < GRP - 1:
        fire_gather(j + 1, nxt, gnxt)
      scale(j, cur)
      fire_scatter(j, cur, scur)
    wait_scatter(GRP - 1, bufs[(GRP - 1) % 2], ssems[(GRP - 1) % 2])

  plsc.subcore_barrier()
  pltpu.sync_copy(acc.at[pl.ds(base, rpt)], out.at[c, pl.ds(base, rpt)])


def _make_agg_kernel(n, e_pad, core_split, kf_scale):
  wsz = 128
  ngroups = NS if core_split else NS * NC
  per_g = e_pad // ngroups
  nwin = per_g // wsz
  n_pad = ((n + NS * 8 - 1) // (NS * 8)) * NS * 8
  mesh = plsc.VectorSubcoreMesh(core_axis_name="c", subcore_axis_name="s")
  return pl.kernel(
      functools.partial(_agg_body, n, core_split, kf_scale, nwin),
      out_type=jax.ShapeDtypeStruct((NC, n_pad, 128), jnp.float32),
      mesh=mesh,
      compiler_params=pltpu.CompilerParams(needs_layout_passes=False,
                                           use_tc_tiling_on_sc=False),
      scratch_types=[
          pltpu.VMEM_SHARED((n_pad, 128), jnp.float32),
          pltpu.VMEM((GRP, wsz), jnp.int32),
          pltpu.VMEM((GRP, wsz), jnp.int32),
          pltpu.VMEM((GRP * wsz,), jnp.float32),
          pltpu.VMEM((wsz, 128), jnp.float32),
          pltpu.VMEM((wsz, 128), jnp.float32),
          pltpu.SemaphoreType.DMA,
          pltpu.SemaphoreType.DMA,
          pltpu.SemaphoreType.DMA,
          pltpu.SemaphoreType.DMA,
      ],
  )


ROWB = 512


def _mm1_body(x_ref, w1_ref, deg2_ref, hs_ref, dinv_ref):
  h = jnp.dot(x_ref[...], w1_ref[...], preferred_element_type=jnp.float32)
  deg = deg2_ref[0, :] + deg2_ref[1, :] + 1.0
  dinv = jnp.where(deg > 0, lax.rsqrt(deg), 0.0)
  hs = h * dinv[:, None]
  f2 = hs.shape[1] // 2
  hs_ref[...] = jnp.stack([hs[:, :f2], hs[:, f2:]])
  dinv_ref[...] = dinv


def _stage3_body(agg_ref, hs_ref, dinv_ref, b1_ref, w2_ref, hs2_ref):
  agg = jnp.concatenate([agg_ref[0], agg_ref[1]], axis=1)
  hs = jnp.concatenate([hs_ref[0], hs_ref[1]], axis=1)
  dinv = dinv_ref[...]
  z = jax.nn.relu(dinv[:, None] * (agg + hs) + b1_ref[0, :][None, :])
  h2 = jnp.dot(z, w2_ref[...], preferred_element_type=jnp.float32)
  hs2 = h2 * dinv[:, None]
  hs2_ref[...] = jnp.concatenate(
      [hs2, jnp.zeros_like(hs2)], axis=1)


def _stage5_body(agg_ref, hs2_ref, dinv_ref, b2_ref, out_ref):
  ncls = out_ref.shape[1]
  agg = agg_ref[0, :, :ncls] + agg_ref[1, :, :ncls]
  hs2 = hs2_ref[:, :ncls]
  dinv = dinv_ref[...]
  logits = dinv[:, None] * (agg + hs2) + b2_ref[0, :][None, :]
  m = jnp.max(logits, axis=1, keepdims=True)
  lse = m + jnp.log(jnp.sum(jnp.exp(logits - m), axis=1, keepdims=True))
  out_ref[...] = logits - lse


def kernel(x, edge_index, edge_weight, W1, b1, W2, b2):
  n, f_in = x.shape
  hid = W1.shape[1]
  ncls = W2.shape[1]
  e = edge_index.shape[1]

  chunk = NC * NS * 128
  e_pad = ((e + chunk - 1) // chunk) * chunk
  pad = e_pad - e
  src = edge_index[0]
  dst = edge_index[1]
  ew = edge_weight
  if pad:
    fill = (jnp.arange(pad, dtype=jnp.int32) * 37) % n
    src = jnp.concatenate([src, fill])
    dst = jnp.concatenate([dst, fill])
    ew = jnp.concatenate([ew, jnp.zeros((pad,), ew.dtype)])

  per_w = e_pad // (NC * NS)
  dst_deg = dst.reshape(NC * NS, per_w // 64, 64)
  ew_deg = ew.reshape(NC * NS, per_w)

  per_s = e_pad // NS
  src_agg1 = src.reshape(NS, per_s // 128, 128)
  dst_agg1 = dst.reshape(NS, per_s // 128, 128)
  ew_agg1 = ew.reshape(NS, per_s)

  src_agg2 = src.reshape(NC * NS, per_w // 128, 128)
  dst_agg2 = dst.reshape(NC * NS, per_w // 128, 128)
  ew_agg2 = ew.reshape(NC * NS, per_w)

  deg2 = _make_deg_kernel(n, e_pad)(dst_deg, ew_deg)

  grid = (n + ROWB - 1) // ROWB
  hs_r, dinv = pl.pallas_call(
      _mm1_body,
      grid=(grid,),
      in_specs=[
          pl.BlockSpec((ROWB, f_in), lambda i: (i, 0)),
          pl.BlockSpec((f_in, hid), lambda i: (0, 0)),
          pl.BlockSpec((NC, ROWB), lambda i: (0, i)),
      ],
      out_specs=[
          pl.BlockSpec((NC, ROWB, hid // 2), lambda i: (0, i, 0)),
          pl.BlockSpec((ROWB,), lambda i: (i,)),
      ],
      out_shape=[
          jax.ShapeDtypeStruct((NC, n, hid // 2), jnp.float32),
          jax.ShapeDtypeStruct((n,), jnp.float32),
      ],
  )(x, W1, deg2)

  tbl1 = hs_r.reshape(NC * n, hid // 2)
  agg1 = _make_agg_kernel(n, e_pad, True, 8)(tbl1, src_agg1, dst_agg1,
                                             ew_agg1)

  hs2p = pl.pallas_call(
      _stage3_body,
      grid=(grid,),
      in_specs=[
          pl.BlockSpec((NC, ROWB, hid // 2), lambda i: (0, i, 0)),
          pl.BlockSpec((NC, ROWB, hid // 2), lambda i: (0, i, 0)),
          pl.BlockSpec((ROWB,), lambda i: (i,)),
          pl.BlockSpec((1, hid), lambda i: (0, 0)),
          pl.BlockSpec((hid, ncls), lambda i: (0, 0)),
      ],
      out_specs=pl.BlockSpec((ROWB, 2 * ncls), lambda i: (i, 0)),
      out_shape=jax.ShapeDtypeStruct((n, 2 * ncls), jnp.float32),
  )(agg1, hs_r, dinv, b1.reshape(1, hid), W2)

  agg2 = _make_agg_kernel(n, e_pad, False, 4)(hs2p, src_agg2, dst_agg2,
                                              ew_agg2)

  out = pl.pallas_call(
      _stage5_body,
      grid=(grid,),
      in_specs=[
          pl.BlockSpec((NC, ROWB, 2 * ncls), lambda i: (0, i, 0)),
          pl.BlockSpec((ROWB, 2 * ncls), lambda i: (i, 0)),
          pl.BlockSpec((ROWB,), lambda i: (i,)),
          pl.BlockSpec((1, ncls), lambda i: (0, 0)),
      ],
      out_specs=pl.BlockSpec((ROWB, ncls), lambda i: (i, 0)),
      out_shape=jax.ShapeDtypeStruct((n, ncls), jnp.float32),
  )(agg2, hs2p, dinv, b2.reshape(1, ncls))

  return out
